# baseline (speedup 1.0000x reference)
import functools

import jax
import jax.numpy as jnp
from jax import lax
from jax.experimental import pallas as pl
from jax.experimental.pallas import tpu as pltpu
from jax.experimental.pallas import tpu_sc as plsc

B, N, C = 4, 4096, 2048

TC_ROWS = 2304
SC_ROWS = N - TC_ROWS


L = 16
NW = 32
SEG = (B * C) // NW
NSEG = C // SEG
R = 128
NCHUNK = SC_ROWS // R
JGRP = SEG // L
HALF = JGRP // 2

_mesh = plsc.VectorSubcoreMesh(core_axis_name="c", subcore_axis_name="s")


@functools.partial(
    pl.kernel,
    mesh=_mesh,
    out_type=[
        jax.ShapeDtypeStruct((B, C), jnp.float32),
        jax.ShapeDtypeStruct((B, C), jnp.int32),
    ],
    scratch_types=[
        pltpu.VMEM((R, SEG), jnp.float32),
        pltpu.VMEM((R, SEG), jnp.float32),
        pltpu.VMEM((SEG,), jnp.float32),
        pltpu.VMEM((SEG,), jnp.int32),
        pltpu.SemaphoreType.DMA,
        pltpu.SemaphoreType.DMA,
    ],
)
def _argmin_sc(x_hbm, val_hbm, idx_hbm, buf0, buf1, minv, mini, sem0, sem1):
    cid = lax.axis_index("c")
    sid = lax.axis_index("s")
    wid = sid * 2 + cid
    b = wid // NSEG
    c0 = (wid % NSEG) * SEG

    def start(g, buf, sem):
        pltpu.async_copy(
            x_hbm.at[b, pl.ds(TC_ROWS + g * R, R), pl.ds(c0, SEG)], buf, sem
        )

    def wait(buf, sem):
        pltpu.make_async_copy(
            x_hbm.at[b, pl.ds(TC_ROWS, R), pl.ds(c0, SEG)], buf, sem
        ).wait()

    for j in range(JGRP):
        minv[pl.ds(j * L, L)] = jnp.full((L,), jnp.inf, jnp.float32)
        mini[pl.ds(j * L, L)] = jnp.zeros((L,), jnp.int32)

    incs = [jnp.full((L,), k, jnp.int32) for k in range(1, 4)]
    four = jnp.full((L,), 4, jnp.int32)

    def compute(g, buf):
        for h in range(2):
            mvs = tuple(minv[pl.ds((h * HALF + jj) * L, L)] for jj in range(HALF))
            mis = tuple(mini[pl.ds((h * HALF + jj) * L, L)] for jj in range(HALF))
            riv0 = jnp.full((L,), TC_ROWS + g * R, jnp.int32)

            def row_body(r4, carry):
                riv, mv, mi = carry
                idxvs = [riv] + [riv + inc for inc in incs]
                mv, mi = list(mv), list(mi)
                for t in range(4):
                    r = r4 * 4 + t
                    for jj in range(HALF):
                        v = buf[r, pl.ds((h * HALF + jj) * L, L)]
                        lt = v < mv[jj]
                        mv[jj] = jnp.where(lt, v, mv[jj])
                        mi[jj] = jnp.where(lt, idxvs[t], mi[jj])
                return riv + four, tuple(mv), tuple(mi)

            _, mvs, mis = lax.fori_loop(0, R // 4, row_body, (riv0, mvs, mis))
            for jj in range(HALF):
                minv[pl.ds((h * HALF + jj) * L, L)] = mvs[jj]
                mini[pl.ds((h * HALF + jj) * L, L)] = mis[jj]

    start(0, buf0, sem0)

    def outer(g2, _):
        for t in range(2):
            g = g2 * 2 + t
            bufc, semc = (buf0, sem0) if t == 0 else (buf1, sem1)
            bufn, semn = (buf1, sem1) if t == 0 else (buf0, sem0)

            @pl.when(g + 1 < NCHUNK)
            def _():
                start(g + 1, bufn, semn)

            wait(bufc, semc)
            compute(g, bufc)
        return 0

    lax.fori_loop(0, NCHUNK // 2, outer, 0)
    pltpu.sync_copy(minv, val_hbm.at[b, pl.ds(c0, SEG)])
    pltpu.sync_copy(mini, idx_hbm.at[b, pl.ds(c0, SEG)])


RC = 256
CB = 512


def _tc_body(x_ref, val_ref, idx_ref):
    rk = pl.program_id(2)
    v = x_ref[0]
    iota = lax.broadcasted_iota(jnp.int32, (RC, CB), 0)
    mv = jnp.min(v, axis=0, keepdims=True)
    hit = v == mv
    mi = jnp.min(jnp.where(hit, iota, RC), axis=0, keepdims=True) + rk * RC

    @pl.when(rk == 0)
    def _():
        val_ref[0] = mv
        idx_ref[0] = mi

    @pl.when(rk != 0)
    def _():
        pv = val_ref[0]
        lt = mv < pv
        val_ref[0] = jnp.where(lt, mv, pv)
        idx_ref[0] = jnp.where(lt, mi, idx_ref[0])


_tc_argmin = pl.pallas_call(
    _tc_body,
    grid=(B, C // CB, TC_ROWS // RC),
    in_specs=[pl.BlockSpec((1, RC, CB), lambda b, cb, rk: (b, rk, cb))],
    out_specs=[
        pl.BlockSpec((1, 1, CB), lambda b, cb, rk: (b, 0, cb)),
        pl.BlockSpec((1, 1, CB), lambda b, cb, rk: (b, 0, cb)),
    ],
    out_shape=[
        jax.ShapeDtypeStruct((B, 1, C), jnp.float32),
        jax.ShapeDtypeStruct((B, 1, C), jnp.int32),
    ],
)


def kernel(x):
    sc_val, sc_idx = _argmin_sc(x)
    tc_val, tc_idx = _tc_argmin(x)
    out = jnp.where(tc_val[:, 0] <= sc_val, tc_idx[:, 0], sc_idx)
    return out.astype(jnp.int64)
